# baseline (speedup 1.0000x reference)
import functools

import jax
import jax.numpy as jnp
import numpy as np
from jax import lax
from jax.experimental import pallas as pl
from jax.experimental.pallas import tpu as pltpu
from jax.experimental.pallas import tpu_sc as plsc

N = 10000
E = 160000
T = 320000
H = 64
TIF = 40
EIF = 80
NL = 2
NG = 2

_NC = 2
_NS = 16
_NW = _NC * _NS
_CH = 40


def _stripes(nseg):
    for ns in (16, 10, 8, 5, 4, 2):
        if nseg % ns == 0 and (nseg // ns) % 8 == 0:
            return ns
    raise ValueError(nseg)


def _ln(x, g, b):
    m = jnp.mean(x, axis=-1, keepdims=True)
    v = jnp.var(x, axis=-1, keepdims=True)
    return (x - m) / jnp.sqrt(v + 1e-5) * g + b


def _mlp(x, W, b, g, be):
    return jax.nn.silu(_ln(x @ W + b, g, be))


def _rbf(d, vmin, vmax, bins):
    centers = jnp.linspace(vmin, vmax, bins)
    gamma = 1.0 / (centers[1] - centers[0]) ** 2
    return jnp.exp(-gamma * (d[:, None] - centers[None, :]) ** 2)


def _bn(x, g, b):
    return x / jnp.sqrt(1.0 + 1e-5) * g + b


def _segsum_body(nseg, n_chunks_w, vals_hbm, idx_hbm, zeros_hbm, out_hbm,
                 idx_v, vbuf, acc, sem0, sem1):
    c = lax.axis_index("c")
    s = lax.axis_index("s")
    w = c * _NS + s
    row0 = w * n_chunks_w * _CH
    nst = _stripes(nseg)
    seg_pw = nseg // nst

    @pl.when(s < nst)
    def _():
        pltpu.sync_copy(zeros_hbm, acc.at[pl.ds(s * seg_pw, seg_pw)])

    pltpu.sync_copy(idx_hbm.at[w], idx_v)
    plsc.subcore_barrier()

    sems = (sem0, sem1)

    def _fire(k, slot):
        pltpu.async_copy(vals_hbm.at[pl.ds(row0 + k * _CH, _CH)],
                         vbuf.at[slot], sems[slot])

    def _wait(k, slot):
        pltpu.make_async_copy(vals_hbm.at[pl.ds(row0 + k * _CH, _CH)],
                              vbuf.at[slot], sems[slot]).wait()

    _fire(0, 0)
    n_pairs = n_chunks_w // 2

    def body(i, carry):
        k = 2 * i
        _wait(k, 0)
        _fire(k + 1, 1)
        pltpu.sync_copy(vbuf.at[0], acc.at[idx_v.at[k]], add=True)
        _wait(k + 1, 1)

        @pl.when(i + 1 < n_pairs)
        def _():
            _fire(k + 2, 0)

        pltpu.sync_copy(vbuf.at[1], acc.at[idx_v.at[k + 1]], add=True)
        return carry

    lax.fori_loop(0, n_pairs, body, 0)

    plsc.subcore_barrier()

    @pl.when(s < nst)
    def _():
        pltpu.sync_copy(acc.at[pl.ds(s * seg_pw, seg_pw)],
                        out_hbm.at[c, pl.ds(s * seg_pw, seg_pw)])


def _segsum_sc(vals, idx3, nseg):
    ev = vals.shape[0]
    n_chunks_w = ev // (_CH * _NW)
    nst = _stripes(nseg)
    zeros = jnp.zeros((nseg // nst, 128), jnp.float32)
    mesh = plsc.VectorSubcoreMesh(core_axis_name="c", subcore_axis_name="s")
    body = functools.partial(_segsum_body, nseg, n_chunks_w)
    k = pl.kernel(
        body,
        out_type=jax.ShapeDtypeStruct((_NC, nseg, 128), jnp.float32),
        mesh=mesh,
        scratch_types=[
            pltpu.VMEM((n_chunks_w, _CH), jnp.int32),
            pltpu.VMEM((2, _CH, 128), jnp.float32),
            pltpu.VMEM_SHARED((nseg, 128), jnp.float32),
            pltpu.SemaphoreType.DMA,
            pltpu.SemaphoreType.DMA,
        ],
    )
    return k(vals, idx3, zeros)


_DEPTH = 8


def _gather_body(n_chunks_w, tab_hbm, idx_hbm, out_hbm, idx_v, vbuf, *sems):
    c = lax.axis_index("c")
    s = lax.axis_index("s")
    w = c * _NS + s
    row0 = w * n_chunks_w * _CH

    pltpu.sync_copy(idx_hbm.at[w], idx_v)
    gsems = sems[:_DEPTH]
    osems = sems[_DEPTH:]

    def _ofs(k):
        return pl.ds(row0 + k * _CH, _CH)

    def _fire_g(k, slot):
        pltpu.async_copy(tab_hbm.at[idx_v.at[k]], vbuf.at[slot], gsems[slot])

    def _wait_g(k, slot):
        pltpu.make_async_copy(tab_hbm.at[idx_v.at[k]], vbuf.at[slot],
                              gsems[slot]).wait()

    def _fire_o(k, slot):
        pltpu.async_copy(vbuf.at[slot], out_hbm.at[_ofs(k)], osems[slot])

    def _wait_o(k, slot):
        pltpu.make_async_copy(vbuf.at[slot], out_hbm.at[_ofs(k)],
                              osems[slot]).wait()

    for slot in range(min(_DEPTH, n_chunks_w)):
        _fire_g(slot, slot)

    n_blocks = (n_chunks_w + _DEPTH - 1) // _DEPTH

    def body(i, carry):
        k0 = i * _DEPTH
        for slot in range(_DEPTH):
            k = k0 + slot

            @pl.when(k < n_chunks_w)
            def _():
                _wait_g(k, slot)
                _fire_o(k, slot)

        for slot in range(_DEPTH):
            k = k0 + slot

            @pl.when(k + _DEPTH < n_chunks_w)
            def _():
                _wait_o(k, slot)
                _fire_g(k + _DEPTH, slot)

        return carry

    lax.fori_loop(0, n_blocks, body, 0)
    for k in range(max(0, n_chunks_w - _DEPTH), n_chunks_w):
        _wait_o(k, k % _DEPTH)


def _gather_sc(tab, idx3):
    n_chunks_w = idx3.shape[1]
    ev = idx3.shape[0] * idx3.shape[1] * idx3.shape[2]
    width = tab.shape[1]
    mesh = plsc.VectorSubcoreMesh(core_axis_name="c", subcore_axis_name="s")
    body = functools.partial(_gather_body, n_chunks_w)
    k = pl.kernel(
        body,
        out_type=jax.ShapeDtypeStruct((ev, width), jnp.float32),
        mesh=mesh,
        scratch_types=[
            pltpu.VMEM((n_chunks_w, _CH), jnp.int32),
            pltpu.VMEM((_DEPTH, _CH, width), jnp.float32),
        ] + [pltpu.SemaphoreType.DMA] * (2 * _DEPTH),
    )
    return k(tab, idx3)


_BLK = 1000


def _rep_spec(shape):
    return pl.BlockSpec(shape, lambda i: (0,) * len(shape))


def _tc_tables_body(x_ref, W_ref, b_ref, ad_ref, bb_ref, x4_ref):
    xb = x_ref[...]
    A = xb @ W_ref[0] + b_ref[0]
    Bt = xb @ W_ref[1] + b_ref[1]
    D = xb @ W_ref[3] + b_ref[3]
    ad_ref[...] = jnp.concatenate([A, D], axis=1)
    bb_ref[...] = jnp.concatenate([Bt, Bt], axis=1)
    x4_ref[...] = xb @ W_ref[4] + b_ref[4]


def _tc_tables(x, W, b):
    n = x.shape[0]
    return pl.pallas_call(
        _tc_tables_body,
        grid=(n // _BLK,),
        in_specs=[
            pl.BlockSpec((_BLK, H), lambda i: (i, 0)),
            _rep_spec((5, H, H)),
            _rep_spec((5, H)),
        ],
        out_specs=[
            pl.BlockSpec((_BLK, 2 * H), lambda i: (i, 0)),
            pl.BlockSpec((_BLK, 2 * H), lambda i: (i, 0)),
            pl.BlockSpec((_BLK, H), lambda i: (i, 0)),
        ],
        out_shape=[
            jax.ShapeDtypeStruct((n, 2 * H), jnp.float32),
            jax.ShapeDtypeStruct((n, 2 * H), jnp.float32),
            jax.ShapeDtypeStruct((n, H), jnp.float32),
        ],
    )(x, W, b)


def _tc_mpass_body(g_ref, bd_ref, y_ref, W2_ref, b2_ref, m_ref, vals_ref):
    C = y_ref[...] @ W2_ref[...] + b2_ref[...]
    m = g_ref[:, :H] + bd_ref[:, :H] + C
    sig = jax.nn.sigmoid(m)
    m_ref[...] = m
    vals_ref[...] = jnp.concatenate([sig * g_ref[:, H:], sig], axis=1)


def _tc_mpass(g, Bd, y, W2, b2):
    ev = g.shape[0]
    return pl.pallas_call(
        _tc_mpass_body,
        grid=(ev // _BLK,),
        in_specs=[
            pl.BlockSpec((_BLK, 2 * H), lambda i: (i, 0)),
            pl.BlockSpec((_BLK, 2 * H), lambda i: (i, 0)),
            pl.BlockSpec((_BLK, H), lambda i: (i, 0)),
            _rep_spec((H, H)),
            _rep_spec((H,)),
        ],
        out_specs=[
            pl.BlockSpec((_BLK, H), lambda i: (i, 0)),
            pl.BlockSpec((_BLK, 2 * H), lambda i: (i, 0)),
        ],
        out_shape=[
            jax.ShapeDtypeStruct((ev, H), jnp.float32),
            jax.ShapeDtypeStruct((ev, 2 * H), jnp.float32),
        ],
    )(g, Bd, y, W2, b2)


def _tc_updx_body(x_ref, x4_ref, p_ref, bg_ref, bb_ref, xo_ref):
    tot = jnp.sum(p_ref[...], axis=0)
    h = tot[:, :H] / (tot[:, H:] + 1e-6)
    t = _bn(x4_ref[...] + h, bg_ref[...], bb_ref[...])
    xo_ref[...] = x_ref[...] + jax.nn.silu(t)


def _tc_updx(x, x4, part, bgv, bbv):
    n = x.shape[0]
    p = part.shape[0]
    return pl.pallas_call(
        _tc_updx_body,
        grid=(n // _BLK,),
        in_specs=[
            pl.BlockSpec((_BLK, H), lambda i: (i, 0)),
            pl.BlockSpec((_BLK, H), lambda i: (i, 0)),
            pl.BlockSpec((p, _BLK, 2 * H), lambda i: (0, i, 0)),
            _rep_spec((H,)),
            _rep_spec((H,)),
        ],
        out_specs=pl.BlockSpec((_BLK, H), lambda i: (i, 0)),
        out_shape=jax.ShapeDtypeStruct((n, H), jnp.float32),
    )(x, x4, part, bgv, bbv)


def _tc_updy_body(y_ref, m_ref, bg_ref, bb_ref, yo_ref):
    t = _bn(m_ref[...], bg_ref[...], bb_ref[...])
    yo_ref[...] = y_ref[...] + jax.nn.silu(t)


def _tc_updy(y, m, bgv, bbv):
    ev = y.shape[0]
    return pl.pallas_call(
        _tc_updy_body,
        grid=(ev // _BLK,),
        in_specs=[
            pl.BlockSpec((_BLK, H), lambda i: (i, 0)),
            pl.BlockSpec((_BLK, H), lambda i: (i, 0)),
            _rep_spec((H,)),
            _rep_spec((H,)),
        ],
        out_specs=pl.BlockSpec((_BLK, H), lambda i: (i, 0)),
        out_shape=jax.ShapeDtypeStruct((ev, H), jnp.float32),
    )(y, m, bgv, bbv)


def _tc_emb2_body(gamma, d_ref, cen_ref, W1_ref, b1_ref, g1_ref, be1_ref,
                  W2_ref, b2_ref, g2_ref, be2_ref, out_ref):
    r = jnp.exp(-gamma * (d_ref[...] - cen_ref[...]) ** 2)
    h1 = _mlp(r, W1_ref[...], b1_ref[...], g1_ref[...], be1_ref[...])
    out_ref[...] = _mlp(h1, W2_ref[...], b2_ref[...], g2_ref[...], be2_ref[...])


def _tc_emb2(d, vmin, vmax, bins, W1, b1, g1, be1, W2, b2, g2, be2):
    ev = d.shape[0]
    centers = np.linspace(vmin, vmax, bins).astype(np.float32)[None, :]
    gamma = float(1.0 / (centers[0, 1] - centers[0, 0]) ** 2)
    body = functools.partial(_tc_emb2_body, gamma)
    return pl.pallas_call(
        body,
        grid=(ev // _BLK,),
        in_specs=[
            pl.BlockSpec((_BLK, 1), lambda i: (i, 0)),
            _rep_spec((1, bins)),
            _rep_spec(W1.shape), _rep_spec(b1.shape),
            _rep_spec(g1.shape), _rep_spec(be1.shape),
            _rep_spec(W2.shape), _rep_spec(b2.shape),
            _rep_spec(g2.shape), _rep_spec(be2.shape),
        ],
        out_specs=pl.BlockSpec((_BLK, H), lambda i: (i, 0)),
        out_shape=jax.ShapeDtypeStruct((ev, H), jnp.float32),
    )(d.reshape(ev, 1), jnp.asarray(centers), W1, b1, g1, be1, W2, b2, g2, be2)


def _tc_emb1_body(x_ref, W_ref, b_ref, g_ref, be_ref, out_ref):
    out_ref[...] = _mlp(x_ref[...], W_ref[...], b_ref[...], g_ref[...],
                        be_ref[...])


def _tc_emb1(x, W, b, g, be):
    n = x.shape[0]
    return pl.pallas_call(
        _tc_emb1_body,
        grid=(n // _BLK,),
        in_specs=[
            pl.BlockSpec((_BLK, x.shape[1]), lambda i: (i, 0)),
            _rep_spec(W.shape), _rep_spec(b.shape),
            _rep_spec(g.shape), _rep_spec(be.shape),
        ],
        out_specs=pl.BlockSpec((_BLK, H), lambda i: (i, 0)),
        out_shape=jax.ShapeDtypeStruct((n, H), jnp.float32),
    )(x, W, b, g, be)


def _eggc_sc(src3, dst, dst3, x, y, W, b, bg, bb, nseg):
    AD, BB, X4 = _tc_tables(x, W, b)
    g = _gather_sc(AD, src3)
    Bd = _gather_sc(BB, dst3)
    m, vals = _tc_mpass(g, Bd, y, W[2], b[2])
    if nseg == N:
        part = _segsum_sc(vals, dst3, nseg)
    else:
        ssh = jax.ops.segment_sum(vals[:, :H], dst, num_segments=nseg)
        ss = jax.ops.segment_sum(vals[:, H:], dst, num_segments=nseg)
        part = jnp.concatenate([ssh, ss], axis=1)[None]
    xo = _tc_updx(x, X4, part, bg[0], bb[0])
    yo = _tc_updy(y, m, bg[1], bb[1])
    return xo, yo


def _final_body(x_ref, fw_ref, fb_ref, o_ref):
    h = jnp.sum(x_ref[...], axis=0) * (1.0 / N)
    o_ref[0] = jnp.sum(h * fw_ref[:, 0]) + fb_ref[0]


def _final(x, fW, fb):
    out = pl.pallas_call(
        _final_body,
        out_shape=jax.ShapeDtypeStruct((1,), jnp.float32),
        out_specs=pl.BlockSpec(memory_space=pltpu.SMEM),
    )(x, fW, fb)
    return jnp.squeeze(out)


def kernel(atom_features, bondlength, angle_features, edge_index, lg_edge_index, aW, ab, ag, abe, eW1, eb1, eg1, ebe1, eW2, eb2, eg2, ebe2, zW1, zb1, zg1, zbe1, zW2, zb2, zg2, zbe2, cW, cb, cbg, cbb, fW, fb):
    src, dst = edge_index[0], edge_index[1]
    lsrc, ldst = lg_edge_index[0], lg_edge_index[1]
    src3 = src.reshape(_NW, E // (_CH * _NW), _CH)
    dst3 = dst.reshape(_NW, E // (_CH * _NW), _CH)
    lsrc3 = lsrc.reshape(_NW, T // (_CH * _NW), _CH)
    ldst3 = ldst.reshape(_NW, T // (_CH * _NW), _CH)
    z = _tc_emb2(angle_features, -1.0, 1.0, TIF, zW1, zb1, zg1, zbe1, zW2, zb2, zg2, zbe2)
    x = _tc_emb1(atom_features, aW, ab, ag, abe)
    y = _tc_emb2(bondlength, 0.0, 8.0, EIF, eW1, eb1, eg1, ebe1, eW2, eb2, eg2, ebe2)
    k = 0
    for _ in range(NL):
        x, m = _eggc_sc(src3, dst, dst3, x, y, cW[k], cb[k], cbg[k], cbb[k], N)
        k += 1
        y, z = _eggc_sc(lsrc3, ldst, ldst3, m, z, cW[k], cb[k], cbg[k], cbb[k], E)
        k += 1
    for _ in range(NG):
        x, y = _eggc_sc(src3, dst, dst3, x, y, cW[k], cb[k], cbg[k], cbb[k], N)
        k += 1
    return _final(x, fW, fb)


# speedup vs baseline: 1.1160x; 1.1160x over previous
import functools

import jax
import jax.numpy as jnp
import numpy as np
from jax import lax
from jax.experimental import pallas as pl
from jax.experimental.pallas import tpu as pltpu
from jax.experimental.pallas import tpu_sc as plsc

N = 10000
E = 160000
T = 320000
H = 64
TIF = 40
EIF = 80
NL = 2
NG = 2

_NC = 2
_NS = 16
_NW = _NC * _NS
_CH = 40


def _stripes(nseg):
    for ns in (16, 10, 8, 5, 4, 2):
        if nseg % ns == 0 and (nseg // ns) % 8 == 0:
            return ns
    raise ValueError(nseg)


def _ln(x, g, b):
    m = jnp.mean(x, axis=-1, keepdims=True)
    v = jnp.var(x, axis=-1, keepdims=True)
    return (x - m) / jnp.sqrt(v + 1e-5) * g + b


def _mlp(x, W, b, g, be):
    return jax.nn.silu(_ln(x @ W + b, g, be))


def _rbf(d, vmin, vmax, bins):
    centers = jnp.linspace(vmin, vmax, bins)
    gamma = 1.0 / (centers[1] - centers[0]) ** 2
    return jnp.exp(-gamma * (d[:, None] - centers[None, :]) ** 2)


def _bn(x, g, b):
    return x / jnp.sqrt(1.0 + 1e-5) * g + b


def _segsum_body(nseg, n_chunks_w, vals_hbm, idx_hbm, zeros_hbm, out_hbm,
                 idx_v, vbuf, acc, sem0, sem1):
    c = lax.axis_index("c")
    s = lax.axis_index("s")
    w = c * _NS + s
    row0 = w * n_chunks_w * _CH
    nst = _stripes(nseg)
    seg_pw = nseg // nst

    @pl.when(s < nst)
    def _():
        pltpu.sync_copy(zeros_hbm, acc.at[pl.ds(s * seg_pw, seg_pw)])

    pltpu.sync_copy(idx_hbm.at[w], idx_v)
    plsc.subcore_barrier()

    sems = (sem0, sem1)

    def _fire(k, slot):
        pltpu.async_copy(vals_hbm.at[pl.ds(row0 + k * _CH, _CH)],
                         vbuf.at[slot], sems[slot])

    def _wait(k, slot):
        pltpu.make_async_copy(vals_hbm.at[pl.ds(row0 + k * _CH, _CH)],
                              vbuf.at[slot], sems[slot]).wait()

    _fire(0, 0)
    n_pairs = n_chunks_w // 2

    def body(i, carry):
        k = 2 * i
        _wait(k, 0)
        _fire(k + 1, 1)
        pltpu.sync_copy(vbuf.at[0], acc.at[idx_v.at[k]], add=True)
        _wait(k + 1, 1)

        @pl.when(i + 1 < n_pairs)
        def _():
            _fire(k + 2, 0)

        pltpu.sync_copy(vbuf.at[1], acc.at[idx_v.at[k + 1]], add=True)
        return carry

    lax.fori_loop(0, n_pairs, body, 0)

    plsc.subcore_barrier()

    @pl.when(s < nst)
    def _():
        pltpu.sync_copy(acc.at[pl.ds(s * seg_pw, seg_pw)],
                        out_hbm.at[c, pl.ds(s * seg_pw, seg_pw)])


def _segsum_sc(vals, idx3, nseg):
    ev = vals.shape[0]
    n_chunks_w = ev // (_CH * _NW)
    nst = _stripes(nseg)
    zeros = jnp.zeros((nseg // nst, 128), jnp.float32)
    mesh = plsc.VectorSubcoreMesh(core_axis_name="c", subcore_axis_name="s")
    body = functools.partial(_segsum_body, nseg, n_chunks_w)
    k = pl.kernel(
        body,
        out_type=jax.ShapeDtypeStruct((_NC, nseg, 128), jnp.float32),
        mesh=mesh,
        scratch_types=[
            pltpu.VMEM((n_chunks_w, _CH), jnp.int32),
            pltpu.VMEM((2, _CH, 128), jnp.float32),
            pltpu.VMEM_SHARED((nseg, 128), jnp.float32),
            pltpu.SemaphoreType.DMA,
            pltpu.SemaphoreType.DMA,
        ],
    )
    return k(vals, idx3, zeros)


_DEPTH = 8


def _gather_body(n_chunks_w, tab_hbm, idx_hbm, out_hbm, idx_v, vbuf, *sems):
    c = lax.axis_index("c")
    s = lax.axis_index("s")
    w = c * _NS + s
    row0 = w * n_chunks_w * _CH

    pltpu.sync_copy(idx_hbm.at[w], idx_v)
    gsems = sems[:_DEPTH]
    osems = sems[_DEPTH:]

    def _ofs(k):
        return pl.ds(row0 + k * _CH, _CH)

    def _fire_g(k, slot):
        pltpu.async_copy(tab_hbm.at[idx_v.at[k]], vbuf.at[slot], gsems[slot])

    def _wait_g(k, slot):
        pltpu.make_async_copy(tab_hbm.at[idx_v.at[k]], vbuf.at[slot],
                              gsems[slot]).wait()

    def _fire_o(k, slot):
        pltpu.async_copy(vbuf.at[slot], out_hbm.at[_ofs(k)], osems[slot])

    def _wait_o(k, slot):
        pltpu.make_async_copy(vbuf.at[slot], out_hbm.at[_ofs(k)],
                              osems[slot]).wait()

    for slot in range(min(_DEPTH, n_chunks_w)):
        _fire_g(slot, slot)

    n_blocks = (n_chunks_w + _DEPTH - 1) // _DEPTH

    def body(i, carry):
        k0 = i * _DEPTH
        for slot in range(_DEPTH):
            k = k0 + slot

            @pl.when(k < n_chunks_w)
            def _():
                _wait_g(k, slot)
                _fire_o(k, slot)

        for slot in range(_DEPTH):
            k = k0 + slot

            @pl.when(k + _DEPTH < n_chunks_w)
            def _():
                _wait_o(k, slot)
                _fire_g(k + _DEPTH, slot)

        return carry

    lax.fori_loop(0, n_blocks, body, 0)
    for k in range(max(0, n_chunks_w - _DEPTH), n_chunks_w):
        _wait_o(k, k % _DEPTH)


def _gather_sc(tab, idx3):
    n_chunks_w = idx3.shape[1]
    ev = idx3.shape[0] * idx3.shape[1] * idx3.shape[2]
    width = tab.shape[1]
    mesh = plsc.VectorSubcoreMesh(core_axis_name="c", subcore_axis_name="s")
    body = functools.partial(_gather_body, n_chunks_w)
    k = pl.kernel(
        body,
        out_type=jax.ShapeDtypeStruct((ev, width), jnp.float32),
        mesh=mesh,
        scratch_types=[
            pltpu.VMEM((n_chunks_w, _CH), jnp.int32),
            pltpu.VMEM((_DEPTH, _CH, width), jnp.float32),
        ] + [pltpu.SemaphoreType.DMA] * (2 * _DEPTH),
    )
    return k(tab, idx3)


_BLK = 2000


def _rep_spec(shape):
    return pl.BlockSpec(shape, lambda i: (0,) * len(shape))


def _tc_tables_body(x_ref, W_ref, b_ref, ad_ref, bb_ref, x4_ref):
    xb = x_ref[...]
    A = xb @ W_ref[0] + b_ref[0]
    Bt = xb @ W_ref[1] + b_ref[1]
    D = xb @ W_ref[3] + b_ref[3]
    ad_ref[...] = jnp.concatenate([A, D], axis=1)
    bb_ref[...] = jnp.concatenate([Bt, Bt], axis=1)
    x4_ref[...] = xb @ W_ref[4] + b_ref[4]


def _tc_tables(x, W, b):
    n = x.shape[0]
    return pl.pallas_call(
        _tc_tables_body,
        grid=(n // _BLK,),
        in_specs=[
            pl.BlockSpec((_BLK, H), lambda i: (i, 0)),
            _rep_spec((5, H, H)),
            _rep_spec((5, H)),
        ],
        out_specs=[
            pl.BlockSpec((_BLK, 2 * H), lambda i: (i, 0)),
            pl.BlockSpec((_BLK, 2 * H), lambda i: (i, 0)),
            pl.BlockSpec((_BLK, H), lambda i: (i, 0)),
        ],
        out_shape=[
            jax.ShapeDtypeStruct((n, 2 * H), jnp.float32),
            jax.ShapeDtypeStruct((n, 2 * H), jnp.float32),
            jax.ShapeDtypeStruct((n, H), jnp.float32),
        ],
    )(x, W, b)


def _tc_mpass_body(g_ref, bd_ref, y_ref, W2_ref, b2_ref, m_ref, vals_ref):
    C = y_ref[...] @ W2_ref[...] + b2_ref[...]
    m = g_ref[:, :H] + bd_ref[:, :H] + C
    sig = jax.nn.sigmoid(m)
    m_ref[...] = m
    vals_ref[...] = jnp.concatenate([sig * g_ref[:, H:], sig], axis=1)


def _tc_mpass(g, Bd, y, W2, b2):
    ev = g.shape[0]
    return pl.pallas_call(
        _tc_mpass_body,
        grid=(ev // _BLK,),
        in_specs=[
            pl.BlockSpec((_BLK, 2 * H), lambda i: (i, 0)),
            pl.BlockSpec((_BLK, 2 * H), lambda i: (i, 0)),
            pl.BlockSpec((_BLK, H), lambda i: (i, 0)),
            _rep_spec((H, H)),
            _rep_spec((H,)),
        ],
        out_specs=[
            pl.BlockSpec((_BLK, H), lambda i: (i, 0)),
            pl.BlockSpec((_BLK, 2 * H), lambda i: (i, 0)),
        ],
        out_shape=[
            jax.ShapeDtypeStruct((ev, H), jnp.float32),
            jax.ShapeDtypeStruct((ev, 2 * H), jnp.float32),
        ],
    )(g, Bd, y, W2, b2)


def _tc_updx_body(x_ref, x4_ref, p_ref, bg_ref, bb_ref, xo_ref):
    tot = jnp.sum(p_ref[...], axis=0)
    h = tot[:, :H] / (tot[:, H:] + 1e-6)
    t = _bn(x4_ref[...] + h, bg_ref[...], bb_ref[...])
    xo_ref[...] = x_ref[...] + jax.nn.silu(t)


def _tc_updx(x, x4, part, bgv, bbv):
    n = x.shape[0]
    p = part.shape[0]
    return pl.pallas_call(
        _tc_updx_body,
        grid=(n // _BLK,),
        in_specs=[
            pl.BlockSpec((_BLK, H), lambda i: (i, 0)),
            pl.BlockSpec((_BLK, H), lambda i: (i, 0)),
            pl.BlockSpec((p, _BLK, 2 * H), lambda i: (0, i, 0)),
            _rep_spec((H,)),
            _rep_spec((H,)),
        ],
        out_specs=pl.BlockSpec((_BLK, H), lambda i: (i, 0)),
        out_shape=jax.ShapeDtypeStruct((n, H), jnp.float32),
    )(x, x4, part, bgv, bbv)


def _tc_updy_body(y_ref, m_ref, bg_ref, bb_ref, yo_ref):
    t = _bn(m_ref[...], bg_ref[...], bb_ref[...])
    yo_ref[...] = y_ref[...] + jax.nn.silu(t)


def _tc_updy(y, m, bgv, bbv):
    ev = y.shape[0]
    return pl.pallas_call(
        _tc_updy_body,
        grid=(ev // _BLK,),
        in_specs=[
            pl.BlockSpec((_BLK, H), lambda i: (i, 0)),
            pl.BlockSpec((_BLK, H), lambda i: (i, 0)),
            _rep_spec((H,)),
            _rep_spec((H,)),
        ],
        out_specs=pl.BlockSpec((_BLK, H), lambda i: (i, 0)),
        out_shape=jax.ShapeDtypeStruct((ev, H), jnp.float32),
    )(y, m, bgv, bbv)


def _tc_emb2_body(gamma, d_ref, cen_ref, W1_ref, b1_ref, g1_ref, be1_ref,
                  W2_ref, b2_ref, g2_ref, be2_ref, out_ref):
    r = jnp.exp(-gamma * (d_ref[...] - cen_ref[...]) ** 2)
    h1 = _mlp(r, W1_ref[...], b1_ref[...], g1_ref[...], be1_ref[...])
    out_ref[...] = _mlp(h1, W2_ref[...], b2_ref[...], g2_ref[...], be2_ref[...])


def _tc_emb2(d, vmin, vmax, bins, W1, b1, g1, be1, W2, b2, g2, be2):
    ev = d.shape[0]
    centers = np.linspace(vmin, vmax, bins).astype(np.float32)[None, :]
    gamma = float(1.0 / (centers[0, 1] - centers[0, 0]) ** 2)
    body = functools.partial(_tc_emb2_body, gamma)
    return pl.pallas_call(
        body,
        grid=(ev // _BLK,),
        in_specs=[
            pl.BlockSpec((_BLK, 1), lambda i: (i, 0)),
            _rep_spec((1, bins)),
            _rep_spec(W1.shape), _rep_spec(b1.shape),
            _rep_spec(g1.shape), _rep_spec(be1.shape),
            _rep_spec(W2.shape), _rep_spec(b2.shape),
            _rep_spec(g2.shape), _rep_spec(be2.shape),
        ],
        out_specs=pl.BlockSpec((_BLK, H), lambda i: (i, 0)),
        out_shape=jax.ShapeDtypeStruct((ev, H), jnp.float32),
    )(d.reshape(ev, 1), jnp.asarray(centers), W1, b1, g1, be1, W2, b2, g2, be2)


def _tc_emb1_body(x_ref, W_ref, b_ref, g_ref, be_ref, out_ref):
    out_ref[...] = _mlp(x_ref[...], W_ref[...], b_ref[...], g_ref[...],
                        be_ref[...])


def _tc_emb1(x, W, b, g, be):
    n = x.shape[0]
    return pl.pallas_call(
        _tc_emb1_body,
        grid=(n // _BLK,),
        in_specs=[
            pl.BlockSpec((_BLK, x.shape[1]), lambda i: (i, 0)),
            _rep_spec(W.shape), _rep_spec(b.shape),
            _rep_spec(g.shape), _rep_spec(be.shape),
        ],
        out_specs=pl.BlockSpec((_BLK, H), lambda i: (i, 0)),
        out_shape=jax.ShapeDtypeStruct((n, H), jnp.float32),
    )(x, W, b, g, be)


def _eggc_sc(src3, dst, dst3, x, y, W, b, bg, bb, nseg):
    AD, BB, X4 = _tc_tables(x, W, b)
    g = _gather_sc(AD, src3)
    Bd = _gather_sc(BB, dst3)
    m, vals = _tc_mpass(g, Bd, y, W[2], b[2])
    if nseg == N:
        part = _segsum_sc(vals, dst3, nseg)
    else:
        ssh = jax.ops.segment_sum(vals[:, :H], dst, num_segments=nseg)
        ss = jax.ops.segment_sum(vals[:, H:], dst, num_segments=nseg)
        part = jnp.concatenate([ssh, ss], axis=1)[None]
    xo = _tc_updx(x, X4, part, bg[0], bb[0])
    yo = _tc_updy(y, m, bg[1], bb[1])
    return xo, yo


def _final_body(x_ref, fw_ref, fb_ref, o_ref):
    h = jnp.sum(x_ref[...], axis=0) * (1.0 / N)
    o_ref[0] = jnp.sum(h * fw_ref[:, 0]) + fb_ref[0]


def _final(x, fW, fb):
    out = pl.pallas_call(
        _final_body,
        out_shape=jax.ShapeDtypeStruct((1,), jnp.float32),
        out_specs=pl.BlockSpec(memory_space=pltpu.SMEM),
    )(x, fW, fb)
    return jnp.squeeze(out)


def kernel(atom_features, bondlength, angle_features, edge_index, lg_edge_index, aW, ab, ag, abe, eW1, eb1, eg1, ebe1, eW2, eb2, eg2, ebe2, zW1, zb1, zg1, zbe1, zW2, zb2, zg2, zbe2, cW, cb, cbg, cbb, fW, fb):
    src, dst = edge_index[0], edge_index[1]
    lsrc, ldst = lg_edge_index[0], lg_edge_index[1]
    src3 = src.reshape(_NW, E // (_CH * _NW), _CH)
    dst3 = dst.reshape(_NW, E // (_CH * _NW), _CH)
    lsrc3 = lsrc.reshape(_NW, T // (_CH * _NW), _CH)
    ldst3 = ldst.reshape(_NW, T // (_CH * _NW), _CH)
    z = _tc_emb2(angle_features, -1.0, 1.0, TIF, zW1, zb1, zg1, zbe1, zW2, zb2, zg2, zbe2)
    x = _tc_emb1(atom_features, aW, ab, ag, abe)
    y = _tc_emb2(bondlength, 0.0, 8.0, EIF, eW1, eb1, eg1, ebe1, eW2, eb2, eg2, ebe2)
    k = 0
    for _ in range(NL):
        x, m = _eggc_sc(src3, dst, dst3, x, y, cW[k], cb[k], cbg[k], cbb[k], N)
        k += 1
        y, z = _eggc_sc(lsrc3, ldst, ldst3, m, z, cW[k], cb[k], cbg[k], cbb[k], E)
        k += 1
    for _ in range(NG):
        x, y = _eggc_sc(src3, dst, dst3, x, y, cW[k], cb[k], cbg[k], cbb[k], N)
        k += 1
    return _final(x, fW, fb)
